# Initial kernel scaffold; baseline (speedup 1.0000x reference)
import jax
import jax.numpy as jnp
from jax.experimental import pallas as pl


def kernel(x, sub_edge_index, node_to_subgraph, params):
    raise NotImplementedError("write your pallas kernel here")


# speedup vs baseline: 7.2862x; 7.2862x over previous
import functools

import jax
import jax.numpy as jnp
from jax import lax
from jax.experimental import pallas as pl
from jax.experimental.pallas import tpu as pltpu
from jax.experimental.pallas import tpu_sc as plsc

DD = 128
NN = 10000
EE = 320000
NSEG = 512
NP = 10240
SP = 528
PAD_SEG = 512

NC = 2
NS = 16
EPT = EE // NS
ECH = 80
ENCH = EPT // ECH
NHALF = NP // NC
NTRASH = NHALF + 8
RPT = NP // (NC * NS)
RCH = 80
RNCH = RPT // RCH
OPT = NP // NS
SPT = SP // NS

def _fill(ref, rows, val):
    vec = jnp.full((16,), val, ref.dtype)
    cols = ref.shape[-1] // 16

    def body(r, _):
        for t in range(cols):
            ref[r, pl.ds(t * 16, 16)] = vec
        return 0

    lax.fori_loop(0, rows, body, 0)


def _sc_edge_scatter(h_hbm, src_hbm, dst_hbm, out_hbm,
                     src_v, dstm_v, buf_a, buf_b, acc, sem_a, sem_b):
    c = lax.axis_index("c")
    s = lax.axis_index("s")
    lo = c * NHALF
    _fill(buf_a, ECH, 0.0)
    base = s * (NHALF // NS)
    for k2 in range(8):
        pltpu.sync_copy(buf_a.at[pl.ds(0, 40)],
                        acc.at[pl.ds(base + k2 * 40, 40)])

    @pl.when(s == 0)
    def _():
        pltpu.sync_copy(buf_a.at[pl.ds(0, 8)], acc.at[pl.ds(NHALF, 8)])
    plsc.subcore_barrier()
    pltpu.sync_copy(src_hbm.at[s], src_v)
    pltpu.sync_copy(dst_hbm.at[c, s], dstm_v)
    pltpu.async_copy(h_hbm.at[src_v.at[0]], buf_a, sem_a)

    def body(i, _):
        j = 2 * i
        jn = j + 1
        jn2 = jnp.minimum(j + 2, ENCH - 1)
        pltpu.make_async_copy(h_hbm.at[src_v.at[j]], buf_a, sem_a).wait()
        pltpu.async_copy(h_hbm.at[src_v.at[jn]], buf_b, sem_b)
        pltpu.sync_copy(buf_a, acc.at[dstm_v.at[j]], add=True)
        pltpu.make_async_copy(h_hbm.at[src_v.at[jn]], buf_b, sem_b).wait()
        pltpu.async_copy(h_hbm.at[src_v.at[jn2]], buf_a, sem_a)
        pltpu.sync_copy(buf_b, acc.at[dstm_v.at[jn]], add=True)
        return 0

    lax.fori_loop(0, ENCH // 2, body, 0)
    pltpu.make_async_copy(h_hbm.at[src_v.at[ENCH - 1]], buf_a, sem_a).wait()
    plsc.subcore_barrier()
    for k2 in range(8):
        r0 = base + k2 * 40
        pltpu.sync_copy(acc.at[pl.ds(r0, 40)], buf_a.at[pl.ds(0, 40)])
        pltpu.sync_copy(buf_a.at[pl.ds(0, 40)], out_hbm.at[pl.ds(lo + r0, 40)])


def _sc_seg_sum(m_hbm, nts_hbm, out_hbm, idx_v, rows_v, buf_s, acc):
    c = lax.axis_index("c")
    s = lax.axis_index("s")
    wid = s * NC + c
    _fill(buf_s, 48, 0.0)
    @pl.when(s < 11)
    def _():
        pltpu.sync_copy(buf_s, acc.at[pl.ds(s * 48, 48)])
    plsc.subcore_barrier()
    pltpu.sync_copy(nts_hbm.at[s, c], idx_v)
    base = wid * RPT
    for k2 in range(RNCH):
        pltpu.sync_copy(m_hbm.at[pl.ds(base + k2 * RCH, RCH)], rows_v)
        pltpu.sync_copy(rows_v, acc.at[idx_v.at[k2]], add=True)
    plsc.subcore_barrier()

    @pl.when(s < 11)
    def _():
        pltpu.sync_copy(acc.at[pl.ds(s * 48, 48)], buf_s)
        pltpu.sync_copy(buf_s, out_hbm.at[c, pl.ds(s * 48, 48)])


def _sc_broadcast_add(m_hbm, vne_hbm, nts_hbm, out_hbm,
                      idx_v, rows_v, gbuf, sem):
    c = lax.axis_index("c")
    s = lax.axis_index("s")
    base = (s * NC + c) * RPT
    pltpu.sync_copy(nts_hbm.at[s, c], idx_v)
    for k2 in range(RNCH):
        pltpu.async_copy(vne_hbm.at[idx_v.at[k2]], gbuf, sem)
        pltpu.sync_copy(m_hbm.at[pl.ds(base + k2 * RCH, RCH)], rows_v)
        pltpu.make_async_copy(vne_hbm.at[idx_v.at[k2]], gbuf, sem).wait()

        def body(r, _):
            for t in range(DD // 16):
                sl = pl.ds(t * 16, 16)
                rows_v[r, sl] = rows_v[r, sl] + gbuf[r, sl]
            return 0

        lax.fori_loop(0, RCH, body, 0)
        pltpu.sync_copy(rows_v, out_hbm.at[pl.ds(base + k2 * RCH, RCH)])


def _sc_seg_max(np_hbm, nts_hbm, out_hbm, idx_v, rows_v, acc_v):
    c = lax.axis_index("c")
    s = lax.axis_index("s")
    base = (s * NC + c) * RPT
    _fill(acc_v, SP, -jnp.inf)
    pltpu.sync_copy(nts_hbm.at[s, c], idx_v)
    for k2 in range(RNCH):
        pltpu.sync_copy(np_hbm.at[pl.ds(base + k2 * RCH, RCH)], rows_v)

        def gbody(g, _):
            idxvec = idx_v[k2 * (RCH // 16) + g, :]
            for l in range(16):
                seg = idxvec[l]
                r = g * 16 + l
                for t in range(DD // 16):
                    sl = pl.ds(t * 16, 16)
                    acc_v[seg, sl] = jnp.maximum(acc_v[seg, sl],
                                                 rows_v[r, sl])
            return 0

        lax.fori_loop(0, RCH // 16, gbody, 0)
    pltpu.sync_copy(acc_v, out_hbm.at[c, s])


@functools.lru_cache(maxsize=None)
def _build_sc():
    mesh = plsc.VectorSubcoreMesh(core_axis_name="c", subcore_axis_name="s")
    f32 = jnp.float32
    edge = pl.kernel(
        _sc_edge_scatter, mesh=mesh,
        out_type=jax.ShapeDtypeStruct((NP, DD), f32),
        scratch_types=[
            pltpu.VMEM((ENCH, ECH), jnp.int32),
            pltpu.VMEM((ENCH, ECH), jnp.int32),
            pltpu.VMEM((ECH, DD), f32),
            pltpu.VMEM((ECH, DD), f32),
            pltpu.VMEM_SHARED((NTRASH, DD), f32),
            pltpu.SemaphoreType.DMA,
            pltpu.SemaphoreType.DMA,
        ])
    seg_sum = pl.kernel(
        _sc_seg_sum, mesh=mesh,
        out_type=jax.ShapeDtypeStruct((NC, SP, DD), f32),
        scratch_types=[
            pltpu.VMEM((RNCH, RCH), jnp.int32),
            pltpu.VMEM((RCH, DD), f32),
            pltpu.VMEM((48, DD), f32),
            pltpu.VMEM_SHARED((SP, DD), f32),
        ])
    bcast = pl.kernel(
        _sc_broadcast_add, mesh=mesh,
        out_type=jax.ShapeDtypeStruct((NP, DD), f32),
        scratch_types=[
            pltpu.VMEM((RNCH, RCH), jnp.int32),
            pltpu.VMEM((RCH, DD), f32),
            pltpu.VMEM((RCH, DD), f32),
            pltpu.SemaphoreType.DMA,
        ])
    seg_max = pl.kernel(
        _sc_seg_max, mesh=mesh,
        out_type=jax.ShapeDtypeStruct((NC, NS, SP, DD), f32),
        scratch_types=[
            pltpu.VMEM((RPT // 16, 16), jnp.int32),
            pltpu.VMEM((RCH, DD), f32),
            pltpu.VMEM((SP, DD), f32),
        ])
    return edge, seg_sum, bcast, seg_max


_BM = 1024
_GRID = NP // _BM


def _row(i):
    return (i, 0)


def _fix(i):
    return (0, 0)


def _tc_gin_body(h, a0, ep, w1, s1, c1, w2, s2, c2, o):
    z = h[...] * ep[...] + a0[...]
    y = jnp.dot(z, w1[...], preferred_element_type=jnp.float32)
    y = jnp.maximum(y * s1[...] + c1[...], 0.0)
    y2 = jnp.dot(y, w2[...], preferred_element_type=jnp.float32)
    o[...] = jnp.maximum(y2 * s2[...] + c2[...], 0.0)


_tc_gin = pl.pallas_call(
    _tc_gin_body,
    grid=(_GRID,),
    in_specs=[pl.BlockSpec((_BM, DD), _row)] * 2
    + [pl.BlockSpec((1, DD), _fix),
       pl.BlockSpec((DD, DD), _fix),
       pl.BlockSpec((1, DD), _fix),
       pl.BlockSpec((1, DD), _fix),
       pl.BlockSpec((DD, DD), _fix),
       pl.BlockSpec((1, DD), _fix),
       pl.BlockSpec((1, DD), _fix)],
    out_specs=pl.BlockSpec((_BM, DD), _row),
    out_shape=jax.ShapeDtypeStruct((NP, DD), jnp.float32),
)


def _tc_vn_body(s0, s1, c0, c1, vne, w1, sv1, cv1, w2, sv2, cv2, o):
    cnt = jnp.maximum(c0[...] + c1[...], 1.0)
    vnet = (s0[...] + s1[...]) / cnt + vne[...]
    a = jnp.dot(vnet, w1[...], preferred_element_type=jnp.float32)
    a = jnp.maximum(a * sv1[...] + cv1[...], 0.0)
    v2 = jnp.dot(a, w2[...], preferred_element_type=jnp.float32)
    o[...] = jnp.maximum(v2 * sv2[...] + cv2[...], 0.0)


_tc_vn = pl.pallas_call(
    _tc_vn_body,
    grid=(1,),
    in_specs=[pl.BlockSpec((SP, DD), _fix),
              pl.BlockSpec((SP, DD), _fix),
              pl.BlockSpec((SP, 1), _fix),
              pl.BlockSpec((SP, 1), _fix),
              pl.BlockSpec((SP, DD), _fix),
              pl.BlockSpec((DD, 2 * DD), _fix),
              pl.BlockSpec((1, 2 * DD), _fix),
              pl.BlockSpec((1, 2 * DD), _fix),
              pl.BlockSpec((2 * DD, DD), _fix),
              pl.BlockSpec((1, DD), _fix),
              pl.BlockSpec((1, DD), _fix)],
    out_specs=pl.BlockSpec((SP, DD), _fix),
    out_shape=jax.ShapeDtypeStruct((SP, DD), jnp.float32),
)


def _tc_gcn_pre_body(h2, d0, wg, hg_o, hg2_o):
    dinv = lax.rsqrt(jnp.maximum(d0[...] + 1.0, 1e-12))
    hg = jnp.dot(h2[...], wg[...], preferred_element_type=jnp.float32)
    hg_o[...] = hg
    hg2_o[...] = hg * dinv


_tc_gcn_pre = pl.pallas_call(
    _tc_gcn_pre_body,
    grid=(_GRID,),
    in_specs=[pl.BlockSpec((_BM, DD), _row),
              pl.BlockSpec((_BM, 1), _row),
              pl.BlockSpec((DD, DD), _fix)],
    out_specs=[pl.BlockSpec((_BM, DD), _row), pl.BlockSpec((_BM, DD), _row)],
    out_shape=[jax.ShapeDtypeStruct((NP, DD), jnp.float32),
               jax.ShapeDtypeStruct((NP, DD), jnp.float32)],
)


def _tc_nodep_body(m0, h2, hg, a0, d0, sB, c3, o):
    dinv = lax.rsqrt(jnp.maximum(d0[...] + 1.0, 1e-12))
    hb = (a0[...] * dinv + hg[...] * dinv * dinv) * sB[...] + c3[...]
    o[...] = m0[...] + h2[...] + hb


_tc_nodep = pl.pallas_call(
    _tc_nodep_body,
    grid=(_GRID,),
    in_specs=[pl.BlockSpec((_BM, DD), _row)] * 4
    + [pl.BlockSpec((_BM, 1), _row),
       pl.BlockSpec((1, DD), _fix),
       pl.BlockSpec((1, DD), _fix)],
    out_specs=pl.BlockSpec((_BM, DD), _row),
    out_shape=jax.ShapeDtypeStruct((NP, DD), jnp.float32),
)


def _tc_final_body(p, o):
    o[...] = jnp.max(p[...], axis=0)[:NSEG]


_tc_final = pl.pallas_call(
    _tc_final_body,
    grid=(1,),
    in_specs=[pl.BlockSpec((NC * NS, SP, DD), lambda i: (0, 0, 0))],
    out_specs=pl.BlockSpec((NSEG, DD), _fix),
    out_shape=jax.ShapeDtypeStruct((NSEG, DD), jnp.float32),
)

_BNS = float(1.0 / (1.0 + 1e-5) ** 0.5)


def kernel(x, sub_edge_index, node_to_subgraph, params):
    src = sub_edge_index[0]
    dst = sub_edge_index[1]
    xp = jnp.pad(x, ((0, NP - NN), (0, 0)))
    src_t = src.reshape(NS, ENCH, ECH)
    dst_t = dst.reshape(NS, ENCH, ECH)
    dstm_t = jnp.stack(
        [jnp.where((dst_t >= c * NHALF) & (dst_t < (c + 1) * NHALF),
                   dst_t - c * NHALF, NHALF) for c in range(NC)])
    nts_pad = jnp.pad(node_to_subgraph, (0, NP - NN),
                      constant_values=PAD_SEG)
    nts_t = nts_pad.reshape(NS, NC, RNCH, RCH)
    nts_g = nts_pad.reshape(NS, NC, RPT // 16, 16)

    def v(a):
        return a.reshape(1, -1)

    gin_p = []
    for i in range(2):
        p = params["gin%d" % i]
        bn = params["bn%d" % i]
        s1 = p["g1"] * _BNS
        c1 = p["b1"] * s1 + p["be1"]
        s2 = bn["g"] * _BNS
        c2 = p["b2"] * s2 + bn["b"]
        ep = jnp.full((1, DD), 1.0 + p["eps"], jnp.float32)
        gin_p.append((ep, p["W1"], v(s1), v(c1), p["W2"], v(s2), v(c2)))
    vn_p = []
    for i in range(2):
        p = params["vn%d" % i]
        sv1 = p["g1"] * _BNS
        cv1 = p["b1"] * sv1 + p["be1"]
        sv2 = p["g2"] * _BNS
        cv2 = p["b2"] * sv2 + p["be2"]
        vn_p.append((p["W1"], v(sv1), v(cv1), p["W2"], v(sv2), v(cv2)))
    sgl = params["bn_last"]["g"] * _BNS
    c3 = params["gcn"]["b"] * sgl + params["bn_last"]["b"]

    vne = jnp.broadcast_to(params["vn_weight"][0], (SP, DD))

    sc_edge, sc_seg_sum, sc_bcast, sc_seg_max = _build_sc()

    ones_np = jnp.ones((NP, DD), jnp.float32)
    deg_col = sc_edge(ones_np, src_t, dstm_t)[:, 0:1]
    cnt_parts = sc_seg_sum(ones_np, nts_t)
    c0 = cnt_parts[0, :, 0:1]
    c1 = cnt_parts[1, :, 0:1]

    h = xp
    m0 = None
    for i in range(2):
        agg = sc_edge(h, src_t, dstm_t)
        ep, w1, s1, cc1, w2, s2, cc2 = gin_p[i]
        m = _tc_gin(h, agg, ep, w1, s1, cc1, w2, s2, cc2)
        if i == 0:
            m0 = m
        sums = sc_seg_sum(m, nts_t)
        w1v, sv1, cv1, w2v, sv2, cv2 = vn_p[i]
        vne = _tc_vn(sums[0], sums[1], c0, c1, vne, w1v, sv1, cv1,
                     w2v, sv2, cv2)
        h = sc_bcast(m, vne, nts_t)

    hg, hg2 = _tc_gcn_pre(h, deg_col, params["gcn"]["W"])
    accg = sc_edge(hg2, src_t, dstm_t)
    node_p = _tc_nodep(m0, h, hg, accg, deg_col, v(sgl), v(c3))
    parts = sc_seg_max(node_p, nts_g)
    return _tc_final(parts.reshape(NC * NS, SP, DD))


# speedup vs baseline: 8.6328x; 1.1848x over previous
import functools

import jax
import jax.numpy as jnp
from jax import lax
from jax.experimental import pallas as pl
from jax.experimental.pallas import tpu as pltpu
from jax.experimental.pallas import tpu_sc as plsc

DD = 128
NN = 10000
EE = 320000
NSEG = 512
NP = 10240
SP = 528
PAD_SEG = 512

NC = 2
NS = 16
EPT = EE // NS
ECH = 125
ENCH = EPT // ECH
NHALF = NP // NC
NTRASH = NHALF + 8
RPT = NP // (NC * NS)
RCH = 80
RNCH = RPT // RCH
OPT = NP // NS
SPT = SP // NS

def _fill(ref, rows, val):
    vec = jnp.full((16,), val, ref.dtype)
    cols = ref.shape[-1] // 16

    def body(r, _):
        for t in range(cols):
            ref[r, pl.ds(t * 16, 16)] = vec
        return 0

    lax.fori_loop(0, rows, body, 0)


def _sc_edge_scatter(h_hbm, src_hbm, dst_hbm, out_hbm,
                     src_v, dstm_v, buf_a, buf_b, acc, sem_a, sem_b):
    c = lax.axis_index("c")
    s = lax.axis_index("s")
    lo = c * NHALF
    _fill(buf_a, ECH, 0.0)
    base = s * (NHALF // NS)
    for k2 in range(8):
        pltpu.sync_copy(buf_a.at[pl.ds(0, 40)],
                        acc.at[pl.ds(base + k2 * 40, 40)])

    @pl.when(s == 0)
    def _():
        pltpu.sync_copy(buf_a.at[pl.ds(0, 8)], acc.at[pl.ds(NHALF, 8)])
    plsc.subcore_barrier()
    pltpu.sync_copy(src_hbm.at[s], src_v)
    pltpu.sync_copy(dst_hbm.at[c, s], dstm_v)
    pltpu.async_copy(h_hbm.at[src_v.at[0]], buf_a, sem_a)

    def body(i, _):
        j = 2 * i
        jn = j + 1
        jn2 = jnp.minimum(j + 2, ENCH - 1)
        pltpu.make_async_copy(h_hbm.at[src_v.at[j]], buf_a, sem_a).wait()
        pltpu.async_copy(h_hbm.at[src_v.at[jn]], buf_b, sem_b)
        pltpu.sync_copy(buf_a, acc.at[dstm_v.at[j]], add=True)
        pltpu.make_async_copy(h_hbm.at[src_v.at[jn]], buf_b, sem_b).wait()
        pltpu.async_copy(h_hbm.at[src_v.at[jn2]], buf_a, sem_a)
        pltpu.sync_copy(buf_b, acc.at[dstm_v.at[jn]], add=True)
        return 0

    lax.fori_loop(0, ENCH // 2, body, 0)
    pltpu.make_async_copy(h_hbm.at[src_v.at[ENCH - 1]], buf_a, sem_a).wait()
    plsc.subcore_barrier()
    for k2 in range(8):
        r0 = base + k2 * 40
        pltpu.sync_copy(acc.at[pl.ds(r0, 40)], buf_a.at[pl.ds(0, 40)])
        pltpu.sync_copy(buf_a.at[pl.ds(0, 40)], out_hbm.at[pl.ds(lo + r0, 40)])


def _sc_seg_sum(m_hbm, nts_hbm, out_hbm, idx_v, rows_v, buf_s, acc):
    c = lax.axis_index("c")
    s = lax.axis_index("s")
    wid = s * NC + c
    _fill(buf_s, 48, 0.0)
    @pl.when(s < 11)
    def _():
        pltpu.sync_copy(buf_s, acc.at[pl.ds(s * 48, 48)])
    plsc.subcore_barrier()
    pltpu.sync_copy(nts_hbm.at[s, c], idx_v)
    base = wid * RPT
    for k2 in range(RNCH):
        pltpu.sync_copy(m_hbm.at[pl.ds(base + k2 * RCH, RCH)], rows_v)
        pltpu.sync_copy(rows_v, acc.at[idx_v.at[k2]], add=True)
    plsc.subcore_barrier()

    @pl.when(s < 11)
    def _():
        pltpu.sync_copy(acc.at[pl.ds(s * 48, 48)], buf_s)
        pltpu.sync_copy(buf_s, out_hbm.at[c, pl.ds(s * 48, 48)])


def _sc_broadcast_add(m_hbm, vne_hbm, nts_hbm, out_hbm,
                      idx_v, rows_v, gbuf, sem):
    c = lax.axis_index("c")
    s = lax.axis_index("s")
    base = (s * NC + c) * RPT
    pltpu.sync_copy(nts_hbm.at[s, c], idx_v)
    for k2 in range(RNCH):
        pltpu.async_copy(vne_hbm.at[idx_v.at[k2]], gbuf, sem)
        pltpu.sync_copy(m_hbm.at[pl.ds(base + k2 * RCH, RCH)], rows_v)
        pltpu.make_async_copy(vne_hbm.at[idx_v.at[k2]], gbuf, sem).wait()

        def body(r, _):
            for t in range(DD // 16):
                sl = pl.ds(t * 16, 16)
                rows_v[r, sl] = rows_v[r, sl] + gbuf[r, sl]
            return 0

        lax.fori_loop(0, RCH, body, 0)
        pltpu.sync_copy(rows_v, out_hbm.at[pl.ds(base + k2 * RCH, RCH)])


def _sc_seg_max(np_hbm, nts_hbm, out_hbm, idx_v, rows_v, acc_v):
    c = lax.axis_index("c")
    s = lax.axis_index("s")
    base = (s * NC + c) * RPT
    _fill(acc_v, SP, -jnp.inf)
    pltpu.sync_copy(nts_hbm.at[s, c], idx_v)
    for k2 in range(RNCH):
        pltpu.sync_copy(np_hbm.at[pl.ds(base + k2 * RCH, RCH)], rows_v)

        def gbody(g, _):
            idxvec = idx_v[k2 * (RCH // 16) + g, :]
            for l in range(16):
                seg = idxvec[l]
                r = g * 16 + l
                for t in range(DD // 16):
                    sl = pl.ds(t * 16, 16)
                    acc_v[seg, sl] = jnp.maximum(acc_v[seg, sl],
                                                 rows_v[r, sl])
            return 0

        lax.fori_loop(0, RCH // 16, gbody, 0)
    pltpu.sync_copy(acc_v, out_hbm.at[c, s])


def _sc_deg_cnt(dstm_hbm, nts_hbm, deg_out, cnt_out,
                dstm_v, idxn_v, ones_v, acc_deg, acc_cnt):
    c = lax.axis_index("c")
    s = lax.axis_index("s")
    lo = c * NHALF
    base = s * (NHALF // NS)
    _fill(ones_v, ECH, 0.0)
    for k2 in range(8):
        pltpu.sync_copy(ones_v.at[pl.ds(0, 40)],
                        acc_deg.at[pl.ds(base + k2 * 40, 40)])

    @pl.when(s == 0)
    def _():
        pltpu.sync_copy(ones_v.at[pl.ds(0, 8)], acc_deg.at[pl.ds(NHALF, 8)])

    @pl.when(s < 11)
    def _():
        pltpu.sync_copy(ones_v.at[pl.ds(0, 48)],
                        acc_cnt.at[pl.ds(s * 48, 48)])
    _fill(ones_v, ECH, 1.0)
    plsc.subcore_barrier()
    pltpu.sync_copy(dstm_hbm.at[c, s], dstm_v)
    pltpu.sync_copy(nts_hbm.at[s, c], idxn_v)

    def body(j, _):
        pltpu.sync_copy(ones_v, acc_deg.at[dstm_v.at[j]], add=True)
        return 0

    lax.fori_loop(0, ENCH, body, 0)
    for k2 in range(RNCH):
        pltpu.sync_copy(ones_v.at[pl.ds(0, RCH)],
                        acc_cnt.at[idxn_v.at[k2]], add=True)
    plsc.subcore_barrier()
    for k2 in range(8):
        r0 = base + k2 * 40
        pltpu.sync_copy(acc_deg.at[pl.ds(r0, 40)], ones_v.at[pl.ds(0, 40)])
        pltpu.sync_copy(ones_v.at[pl.ds(0, 40)],
                        deg_out.at[pl.ds(lo + r0, 40)])

    @pl.when(s < 11)
    def _():
        pltpu.sync_copy(acc_cnt.at[pl.ds(s * 48, 48)], ones_v.at[pl.ds(0, 48)])
        pltpu.sync_copy(ones_v.at[pl.ds(0, 48)],
                        cnt_out.at[c, pl.ds(s * 48, 48)])


@functools.lru_cache(maxsize=None)
def _build_sc():
    mesh = plsc.VectorSubcoreMesh(core_axis_name="c", subcore_axis_name="s")
    f32 = jnp.float32
    edge = pl.kernel(
        _sc_edge_scatter, mesh=mesh,
        out_type=jax.ShapeDtypeStruct((NP, DD), f32),
        scratch_types=[
            pltpu.VMEM((ENCH, ECH), jnp.int32),
            pltpu.VMEM((ENCH, ECH), jnp.int32),
            pltpu.VMEM((ECH, DD), f32),
            pltpu.VMEM((ECH, DD), f32),
            pltpu.VMEM_SHARED((NTRASH, DD), f32),
            pltpu.SemaphoreType.DMA,
            pltpu.SemaphoreType.DMA,
        ])
    seg_sum = pl.kernel(
        _sc_seg_sum, mesh=mesh,
        out_type=jax.ShapeDtypeStruct((NC, SP, DD), f32),
        scratch_types=[
            pltpu.VMEM((RNCH, RCH), jnp.int32),
            pltpu.VMEM((RCH, DD), f32),
            pltpu.VMEM((48, DD), f32),
            pltpu.VMEM_SHARED((SP, DD), f32),
        ])
    bcast = pl.kernel(
        _sc_broadcast_add, mesh=mesh,
        out_type=jax.ShapeDtypeStruct((NP, DD), f32),
        scratch_types=[
            pltpu.VMEM((RNCH, RCH), jnp.int32),
            pltpu.VMEM((RCH, DD), f32),
            pltpu.VMEM((RCH, DD), f32),
            pltpu.SemaphoreType.DMA,
        ])
    deg_cnt = pl.kernel(
        _sc_deg_cnt, mesh=mesh,
        out_type=(jax.ShapeDtypeStruct((NP, DD), f32),
                  jax.ShapeDtypeStruct((NC, SP, DD), f32)),
        scratch_types=[
            pltpu.VMEM((ENCH, ECH), jnp.int32),
            pltpu.VMEM((RNCH, RCH), jnp.int32),
            pltpu.VMEM((ECH, DD), f32),
            pltpu.VMEM_SHARED((NTRASH, DD), f32),
            pltpu.VMEM_SHARED((SP, DD), f32),
        ])
    seg_max = pl.kernel(
        _sc_seg_max, mesh=mesh,
        out_type=jax.ShapeDtypeStruct((NC, NS, SP, DD), f32),
        scratch_types=[
            pltpu.VMEM((RPT // 16, 16), jnp.int32),
            pltpu.VMEM((RCH, DD), f32),
            pltpu.VMEM((SP, DD), f32),
        ])
    return edge, seg_sum, bcast, seg_max, deg_cnt


_BM = 1024
_GRID = NP // _BM


def _row(i):
    return (i, 0)


def _fix(i):
    return (0, 0)


def _tc_gin_body(h, a0, ep, w1, s1, c1, w2, s2, c2, o):
    z = h[...] * ep[...] + a0[...]
    y = jnp.dot(z, w1[...], preferred_element_type=jnp.float32)
    y = jnp.maximum(y * s1[...] + c1[...], 0.0)
    y2 = jnp.dot(y, w2[...], preferred_element_type=jnp.float32)
    o[...] = jnp.maximum(y2 * s2[...] + c2[...], 0.0)


_tc_gin = pl.pallas_call(
    _tc_gin_body,
    grid=(_GRID,),
    in_specs=[pl.BlockSpec((_BM, DD), _row)] * 2
    + [pl.BlockSpec((1, DD), _fix),
       pl.BlockSpec((DD, DD), _fix),
       pl.BlockSpec((1, DD), _fix),
       pl.BlockSpec((1, DD), _fix),
       pl.BlockSpec((DD, DD), _fix),
       pl.BlockSpec((1, DD), _fix),
       pl.BlockSpec((1, DD), _fix)],
    out_specs=pl.BlockSpec((_BM, DD), _row),
    out_shape=jax.ShapeDtypeStruct((NP, DD), jnp.float32),
)


def _tc_vn_body(s0, s1, c0, c1, vne, w1, sv1, cv1, w2, sv2, cv2, o):
    cnt = jnp.maximum(c0[...] + c1[...], 1.0)
    vnet = (s0[...] + s1[...]) / cnt + vne[...]
    a = jnp.dot(vnet, w1[...], preferred_element_type=jnp.float32)
    a = jnp.maximum(a * sv1[...] + cv1[...], 0.0)
    v2 = jnp.dot(a, w2[...], preferred_element_type=jnp.float32)
    o[...] = jnp.maximum(v2 * sv2[...] + cv2[...], 0.0)


_tc_vn = pl.pallas_call(
    _tc_vn_body,
    grid=(1,),
    in_specs=[pl.BlockSpec((SP, DD), _fix),
              pl.BlockSpec((SP, DD), _fix),
              pl.BlockSpec((SP, 1), _fix),
              pl.BlockSpec((SP, 1), _fix),
              pl.BlockSpec((SP, DD), _fix),
              pl.BlockSpec((DD, 2 * DD), _fix),
              pl.BlockSpec((1, 2 * DD), _fix),
              pl.BlockSpec((1, 2 * DD), _fix),
              pl.BlockSpec((2 * DD, DD), _fix),
              pl.BlockSpec((1, DD), _fix),
              pl.BlockSpec((1, DD), _fix)],
    out_specs=pl.BlockSpec((SP, DD), _fix),
    out_shape=jax.ShapeDtypeStruct((SP, DD), jnp.float32),
)


def _tc_gcn_pre_body(h2, d0, wg, hg_o, hg2_o):
    dinv = lax.rsqrt(jnp.maximum(d0[...] + 1.0, 1e-12))
    hg = jnp.dot(h2[...], wg[...], preferred_element_type=jnp.float32)
    hg_o[...] = hg
    hg2_o[...] = hg * dinv


_tc_gcn_pre = pl.pallas_call(
    _tc_gcn_pre_body,
    grid=(_GRID,),
    in_specs=[pl.BlockSpec((_BM, DD), _row),
              pl.BlockSpec((_BM, 1), _row),
              pl.BlockSpec((DD, DD), _fix)],
    out_specs=[pl.BlockSpec((_BM, DD), _row), pl.BlockSpec((_BM, DD), _row)],
    out_shape=[jax.ShapeDtypeStruct((NP, DD), jnp.float32),
               jax.ShapeDtypeStruct((NP, DD), jnp.float32)],
)


def _tc_nodep_body(m0, h2, hg, a0, d0, sB, c3, o):
    dinv = lax.rsqrt(jnp.maximum(d0[...] + 1.0, 1e-12))
    hb = (a0[...] * dinv + hg[...] * dinv * dinv) * sB[...] + c3[...]
    o[...] = m0[...] + h2[...] + hb


_tc_nodep = pl.pallas_call(
    _tc_nodep_body,
    grid=(_GRID,),
    in_specs=[pl.BlockSpec((_BM, DD), _row)] * 4
    + [pl.BlockSpec((_BM, 1), _row),
       pl.BlockSpec((1, DD), _fix),
       pl.BlockSpec((1, DD), _fix)],
    out_specs=pl.BlockSpec((_BM, DD), _row),
    out_shape=jax.ShapeDtypeStruct((NP, DD), jnp.float32),
)


def _tc_final_body(p, o):
    o[...] = jnp.max(p[...], axis=0)[:NSEG]


_tc_final = pl.pallas_call(
    _tc_final_body,
    grid=(1,),
    in_specs=[pl.BlockSpec((NC * NS, SP, DD), lambda i: (0, 0, 0))],
    out_specs=pl.BlockSpec((NSEG, DD), _fix),
    out_shape=jax.ShapeDtypeStruct((NSEG, DD), jnp.float32),
)

_BNS = float(1.0 / (1.0 + 1e-5) ** 0.5)


def kernel(x, sub_edge_index, node_to_subgraph, params):
    src = sub_edge_index[0]
    dst = sub_edge_index[1]
    xp = jnp.pad(x, ((0, NP - NN), (0, 0)))
    src_t = src.reshape(NS, ENCH, ECH)
    dst_t = dst.reshape(NS, ENCH, ECH)
    dstm_t = jnp.stack(
        [jnp.where((dst_t >= c * NHALF) & (dst_t < (c + 1) * NHALF),
                   dst_t - c * NHALF, NHALF) for c in range(NC)])
    nts_pad = jnp.pad(node_to_subgraph, (0, NP - NN),
                      constant_values=PAD_SEG)
    nts_t = nts_pad.reshape(NS, NC, RNCH, RCH)
    nts_g = nts_pad.reshape(NS, NC, RPT // 16, 16)

    def v(a):
        return a.reshape(1, -1)

    gin_p = []
    for i in range(2):
        p = params["gin%d" % i]
        bn = params["bn%d" % i]
        s1 = p["g1"] * _BNS
        c1 = p["b1"] * s1 + p["be1"]
        s2 = bn["g"] * _BNS
        c2 = p["b2"] * s2 + bn["b"]
        ep = jnp.full((1, DD), 1.0 + p["eps"], jnp.float32)
        gin_p.append((ep, p["W1"], v(s1), v(c1), p["W2"], v(s2), v(c2)))
    vn_p = []
    for i in range(2):
        p = params["vn%d" % i]
        sv1 = p["g1"] * _BNS
        cv1 = p["b1"] * sv1 + p["be1"]
        sv2 = p["g2"] * _BNS
        cv2 = p["b2"] * sv2 + p["be2"]
        vn_p.append((p["W1"], v(sv1), v(cv1), p["W2"], v(sv2), v(cv2)))
    sgl = params["bn_last"]["g"] * _BNS
    c3 = params["gcn"]["b"] * sgl + params["bn_last"]["b"]

    vne = jnp.broadcast_to(params["vn_weight"][0], (SP, DD))

    sc_edge, sc_seg_sum, sc_bcast, sc_seg_max, sc_deg_cnt = _build_sc()

    deg_full, cnt_parts = sc_deg_cnt(dstm_t, nts_t)
    deg_col = deg_full[:, 0:1]
    c0 = cnt_parts[0, :, 0:1]
    c1 = cnt_parts[1, :, 0:1]

    h = xp
    m0 = None
    for i in range(2):
        agg = sc_edge(h, src_t, dstm_t)
        ep, w1, s1, cc1, w2, s2, cc2 = gin_p[i]
        m = _tc_gin(h, agg, ep, w1, s1, cc1, w2, s2, cc2)
        if i == 0:
            m0 = m
        sums = sc_seg_sum(m, nts_t)
        w1v, sv1, cv1, w2v, sv2, cv2 = vn_p[i]
        vne = _tc_vn(sums[0], sums[1], c0, c1, vne, w1v, sv1, cv1,
                     w2v, sv2, cv2)
        h = sc_bcast(m, vne, nts_t)

    hg, hg2 = _tc_gcn_pre(h, deg_col, params["gcn"]["W"])
    accg = sc_edge(hg2, src_t, dstm_t)
    node_p = _tc_nodep(m0, h, hg, accg, deg_col, v(sgl), v(c3))
    parts = sc_seg_max(node_p, nts_g)
    return _tc_final(parts.reshape(NC * NS, SP, DD))


# speedup vs baseline: 12.0788x; 1.3992x over previous
import functools

import jax
import jax.numpy as jnp
from jax import lax
from jax.experimental import pallas as pl
from jax.experimental.pallas import tpu as pltpu
from jax.experimental.pallas import tpu_sc as plsc

DD = 128
DH = 64
NN = 10000
EE = 320000
NSEG = 512
NP = 10240
NP2 = 2 * NP
SP = 528
SP2 = 2 * SP
PAD_SEG = 512

NC = 2
NS = 16
EPT = EE // NS
ECH = 100
ENCH = EPT // ECH
EPC = EE // (NC * NS)
RPT = NP // NS
RCH = 80
RNCH = RPT // RCH


def _fill(ref, rows, val):
    vec = jnp.full((16,), val, ref.dtype)
    cols = ref.shape[-1] // 16

    def body(r, _):
        for t in range(cols):
            ref[r, pl.ds(t * 16, 16)] = vec
        return 0

    lax.fori_loop(0, rows, body, 0)


def _sc_edge_scatter(h_hbm, src_hbm, dst_hbm, out_hbm,
                     src_v, dst_v, buf_a, buf_b, acc,
                     gs_a, gs_b, ss_a, ss_b):
    c = lax.axis_index("c")
    s = lax.axis_index("s")
    bufs = (buf_a, buf_b)
    gsem = (gs_a, gs_b)
    ssem = (ss_a, ss_b)
    _fill(buf_a, 40, 0.0)
    base = s * RPT
    for k2 in range(RPT // 40):
        pltpu.sync_copy(buf_a.at[pl.ds(0, 40)],
                        acc.at[pl.ds(base + k2 * 40, 40)])
    plsc.subcore_barrier()
    pltpu.sync_copy(src_hbm.at[c, s], src_v)
    pltpu.sync_copy(dst_hbm.at[s], dst_v)

    def body(i, _):
        for t in range(2):
            jj = 2 * i + t
            t2 = (t + 1) % 2

            @pl.when(jj >= 2)
            def _():
                pltpu.make_async_copy(
                    bufs[t], acc.at[dst_v.at[0]], ssem[t]).wait()
            pltpu.async_copy(h_hbm.at[src_v.at[jj]], bufs[t], gsem[t])

            @pl.when(jj >= 1)
            def _():
                pltpu.make_async_copy(
                    h_hbm.at[src_v.at[jj - 1]], bufs[t2], gsem[t2]).wait()
                pltpu.async_copy(bufs[t2], acc.at[dst_v.at[jj - 1]],
                                 ssem[t2], add=True)
        return 0

    lax.fori_loop(0, ENCH // 2, body, 0)
    pltpu.make_async_copy(h_hbm.at[src_v.at[ENCH - 1]], bufs[1],
                          gsem[1]).wait()
    pltpu.async_copy(bufs[1], acc.at[dst_v.at[ENCH - 1]], ssem[1], add=True)
    pltpu.make_async_copy(bufs[0], acc.at[dst_v.at[0]], ssem[0]).wait()
    pltpu.make_async_copy(bufs[1], acc.at[dst_v.at[0]], ssem[1]).wait()
    plsc.subcore_barrier()
    for k2 in range(RPT // 40):
        r0 = base + k2 * 40
        pltpu.sync_copy(acc.at[pl.ds(r0, 40)], buf_a.at[pl.ds(0, 40)])
        pltpu.sync_copy(buf_a.at[pl.ds(0, 40)],
                        out_hbm.at[pl.ds(c * NP + r0, 40)])


def _sc_seg_sum(m_hbm, nts_hbm, out_hbm, idx_v, rows_v, buf_s, acc):
    c = lax.axis_index("c")
    s = lax.axis_index("s")
    _fill(buf_s, 48, 0.0)
    @pl.when(s < 11)
    def _():
        pltpu.sync_copy(buf_s, acc.at[pl.ds(s * 48, 48)])
    plsc.subcore_barrier()
    pltpu.sync_copy(nts_hbm.at[s], idx_v)
    base = c * NP + s * RPT
    for k2 in range(RNCH):
        pltpu.sync_copy(m_hbm.at[pl.ds(base + k2 * RCH, RCH)], rows_v)
        pltpu.sync_copy(rows_v, acc.at[idx_v.at[k2]], add=True)
    plsc.subcore_barrier()

    @pl.when(s < 11)
    def _():
        pltpu.sync_copy(acc.at[pl.ds(s * 48, 48)], buf_s)
        pltpu.sync_copy(buf_s, out_hbm.at[c, pl.ds(s * 48, 48)])


def _sc_broadcast_add(m_hbm, vne_hbm, nts_hbm, out_hbm,
                      idx_v, rows_v, gbuf, sem):
    c = lax.axis_index("c")
    s = lax.axis_index("s")
    base = c * NP + s * RPT
    pltpu.sync_copy(nts_hbm.at[c, s], idx_v)
    for k2 in range(RNCH):
        pltpu.async_copy(vne_hbm.at[idx_v.at[k2]], gbuf, sem)
        pltpu.sync_copy(m_hbm.at[pl.ds(base + k2 * RCH, RCH)], rows_v)
        pltpu.make_async_copy(vne_hbm.at[idx_v.at[k2]], gbuf, sem).wait()

        def body(r, _):
            for t in range(DH // 16):
                sl = pl.ds(t * 16, 16)
                rows_v[r, sl] = rows_v[r, sl] + gbuf[r, sl]
            return 0

        lax.fori_loop(0, RCH, body, 0)
        pltpu.sync_copy(rows_v, out_hbm.at[pl.ds(base + k2 * RCH, RCH)])


def _sc_seg_max(np_hbm, nts_hbm, out_hbm, idx_v, rows_v, acc_v):
    c = lax.axis_index("c")
    s = lax.axis_index("s")
    base = c * NP + s * RPT
    _fill(acc_v, SP, -jnp.inf)
    pltpu.sync_copy(nts_hbm.at[s], idx_v)
    for k2 in range(RNCH):
        pltpu.sync_copy(np_hbm.at[pl.ds(base + k2 * RCH, RCH)], rows_v)

        def gbody(g, _):
            idxvec = idx_v[k2 * (RCH // 16) + g, :]
            for l in range(16):
                seg = idxvec[l]
                r = g * 16 + l
                for t in range(DH // 16):
                    sl = pl.ds(t * 16, 16)
                    acc_v[seg, sl] = jnp.maximum(acc_v[seg, sl],
                                                 rows_v[r, sl])
            return 0

        lax.fori_loop(0, RCH // 16, gbody, 0)
    pltpu.sync_copy(acc_v, out_hbm.at[c, s])


def _sc_deg_cnt(dst_hbm, nts_hbm, deg_out, cnt_out,
                dst_v, idxn_v, ones_v, acc_deg, acc_cnt):
    c = lax.axis_index("c")
    s = lax.axis_index("s")
    base = s * RPT
    _fill(ones_v, ECH, 0.0)
    for k2 in range(RPT // 40):
        pltpu.sync_copy(ones_v.at[pl.ds(0, 40)],
                        acc_deg.at[pl.ds(base + k2 * 40, 40)])

    @pl.when(s < 11)
    def _():
        pltpu.sync_copy(ones_v.at[pl.ds(0, 48)],
                        acc_cnt.at[pl.ds(s * 48, 48)])
    _fill(ones_v, ECH, 1.0)
    plsc.subcore_barrier()
    pltpu.sync_copy(dst_hbm.at[c, s], dst_v)
    pltpu.sync_copy(nts_hbm.at[s, c], idxn_v)

    def body(j, _):
        pltpu.sync_copy(ones_v, acc_deg.at[dst_v.at[j]], add=True)
        return 0

    lax.fori_loop(0, EPC // ECH, body, 0)
    for k2 in range(4):
        pltpu.sync_copy(ones_v.at[pl.ds(0, RCH)],
                        acc_cnt.at[idxn_v.at[k2]], add=True)
    plsc.subcore_barrier()
    for k2 in range(RPT // 40):
        r0 = base + k2 * 40
        pltpu.sync_copy(acc_deg.at[pl.ds(r0, 40)], ones_v.at[pl.ds(0, 40)])
        pltpu.sync_copy(ones_v.at[pl.ds(0, 40)],
                        deg_out.at[pl.ds(c * NP + r0, 40)])

    @pl.when(s < 11)
    def _():
        pltpu.sync_copy(acc_cnt.at[pl.ds(s * 48, 48)], ones_v.at[pl.ds(0, 48)])
        pltpu.sync_copy(ones_v.at[pl.ds(0, 48)],
                        cnt_out.at[c, pl.ds(s * 48, 48)])


@functools.lru_cache(maxsize=None)
def _build_sc():
    mesh = plsc.VectorSubcoreMesh(core_axis_name="c", subcore_axis_name="s")
    cp = pltpu.CompilerParams(use_tc_tiling_on_sc=False)
    f32 = jnp.float32
    edge = pl.kernel(
        _sc_edge_scatter, mesh=mesh, compiler_params=cp,
        out_type=jax.ShapeDtypeStruct((NP2, DH), f32),
        scratch_types=[
            pltpu.VMEM((ENCH, ECH), jnp.int32),
            pltpu.VMEM((ENCH, ECH), jnp.int32),
            pltpu.VMEM((ECH, DH), f32),
            pltpu.VMEM((ECH, DH), f32),
            pltpu.VMEM_SHARED((NP, DH), f32),
            pltpu.SemaphoreType.DMA,
            pltpu.SemaphoreType.DMA,
            pltpu.SemaphoreType.DMA,
            pltpu.SemaphoreType.DMA,
        ])
    seg_sum = pl.kernel(
        _sc_seg_sum, mesh=mesh, compiler_params=cp,
        out_type=jax.ShapeDtypeStruct((NC, SP, DH), f32),
        scratch_types=[
            pltpu.VMEM((RNCH, RCH), jnp.int32),
            pltpu.VMEM((RCH, DH), f32),
            pltpu.VMEM((48, DH), f32),
            pltpu.VMEM_SHARED((SP, DH), f32),
        ])
    bcast = pl.kernel(
        _sc_broadcast_add, mesh=mesh, compiler_params=cp,
        out_type=jax.ShapeDtypeStruct((NP2, DH), f32),
        scratch_types=[
            pltpu.VMEM((RNCH, RCH), jnp.int32),
            pltpu.VMEM((RCH, DH), f32),
            pltpu.VMEM((RCH, DH), f32),
            pltpu.SemaphoreType.DMA,
        ])
    seg_max = pl.kernel(
        _sc_seg_max, mesh=mesh, compiler_params=cp,
        out_type=jax.ShapeDtypeStruct((NC, NS, SP, DH), f32),
        scratch_types=[
            pltpu.VMEM((RPT // 16, 16), jnp.int32),
            pltpu.VMEM((RCH, DH), f32),
            pltpu.VMEM((SP, DH), f32),
        ])
    deg_cnt = pl.kernel(
        _sc_deg_cnt, mesh=mesh, compiler_params=cp,
        out_type=(jax.ShapeDtypeStruct((NP2, DH), f32),
                  jax.ShapeDtypeStruct((NC, SP, DH), f32)),
        scratch_types=[
            pltpu.VMEM((EPC // ECH, ECH), jnp.int32),
            pltpu.VMEM((4, RCH), jnp.int32),
            pltpu.VMEM((ECH, DH), f32),
            pltpu.VMEM_SHARED((NP, DH), f32),
            pltpu.VMEM_SHARED((SP, DH), f32),
        ])
    return edge, seg_sum, bcast, seg_max, deg_cnt


_BM = 1024
_GRID = NP // _BM


def _row(i):
    return (i, 0)


def _fix(i):
    return (0, 0)


def _cat(lo, hi):
    return jnp.concatenate([lo, hi], axis=1)


def _tc_gin_body(hl, hh, al, ah, ep, w1, s1, c1, w2, s2, c2, ol, oh):
    z = _cat(hl[...], hh[...]) * ep[...] + _cat(al[...], ah[...])
    y = jnp.dot(z, w1[...], preferred_element_type=jnp.float32)
    y = jnp.maximum(y * s1[...] + c1[...], 0.0)
    y2 = jnp.dot(y, w2[...], preferred_element_type=jnp.float32)
    m = jnp.maximum(y2 * s2[...] + c2[...], 0.0)
    ol[...] = m[:, :DH]
    oh[...] = m[:, DH:]


_tc_gin = pl.pallas_call(
    _tc_gin_body,
    grid=(_GRID,),
    in_specs=[pl.BlockSpec((_BM, DH), _row)] * 4
    + [pl.BlockSpec((1, DD), _fix),
       pl.BlockSpec((DD, DD), _fix),
       pl.BlockSpec((1, DD), _fix),
       pl.BlockSpec((1, DD), _fix),
       pl.BlockSpec((DD, DD), _fix),
       pl.BlockSpec((1, DD), _fix),
       pl.BlockSpec((1, DD), _fix)],
    out_specs=[pl.BlockSpec((_BM, DH), _row), pl.BlockSpec((_BM, DH), _row)],
    out_shape=[jax.ShapeDtypeStruct((NP, DH), jnp.float32),
               jax.ShapeDtypeStruct((NP, DH), jnp.float32)],
)


def _tc_vn_body(s0, s1, c0, c1, vl, vh, w1, sv1, cv1, w2, sv2, cv2, ol, oh):
    cnt = jnp.maximum(c0[...] + c1[...], 1.0)
    vnet = _cat(s0[...], s1[...]) / cnt + _cat(vl[...], vh[...])
    a = jnp.dot(vnet, w1[...], preferred_element_type=jnp.float32)
    a = jnp.maximum(a * sv1[...] + cv1[...], 0.0)
    v2 = jnp.dot(a, w2[...], preferred_element_type=jnp.float32)
    vn = jnp.maximum(v2 * sv2[...] + cv2[...], 0.0)
    ol[...] = vn[:, :DH]
    oh[...] = vn[:, DH:]


_tc_vn = pl.pallas_call(
    _tc_vn_body,
    grid=(1,),
    in_specs=[pl.BlockSpec((SP, DH), _fix),
              pl.BlockSpec((SP, DH), _fix),
              pl.BlockSpec((SP, 1), _fix),
              pl.BlockSpec((SP, 1), _fix),
              pl.BlockSpec((SP, DH), _fix),
              pl.BlockSpec((SP, DH), _fix),
              pl.BlockSpec((DD, 2 * DD), _fix),
              pl.BlockSpec((1, 2 * DD), _fix),
              pl.BlockSpec((1, 2 * DD), _fix),
              pl.BlockSpec((2 * DD, DD), _fix),
              pl.BlockSpec((1, DD), _fix),
              pl.BlockSpec((1, DD), _fix)],
    out_specs=[pl.BlockSpec((SP, DH), _fix), pl.BlockSpec((SP, DH), _fix)],
    out_shape=[jax.ShapeDtypeStruct((SP, DH), jnp.float32),
               jax.ShapeDtypeStruct((SP, DH), jnp.float32)],
)


def _tc_gcn_pre_body(hl, hh, d0, wg, hgl_o, hgh_o, h2l_o, h2h_o):
    dinv = lax.rsqrt(jnp.maximum(d0[...] + 1.0, 1e-12))
    hg = jnp.dot(_cat(hl[...], hh[...]), wg[...],
                 preferred_element_type=jnp.float32)
    hg2 = hg * dinv
    hgl_o[...] = hg[:, :DH]
    hgh_o[...] = hg[:, DH:]
    h2l_o[...] = hg2[:, :DH]
    h2h_o[...] = hg2[:, DH:]


_tc_gcn_pre = pl.pallas_call(
    _tc_gcn_pre_body,
    grid=(_GRID,),
    in_specs=[pl.BlockSpec((_BM, DH), _row),
              pl.BlockSpec((_BM, DH), _row),
              pl.BlockSpec((_BM, 1), _row),
              pl.BlockSpec((DD, DD), _fix)],
    out_specs=[pl.BlockSpec((_BM, DH), _row)] * 4,
    out_shape=[jax.ShapeDtypeStruct((NP, DH), jnp.float32)] * 4,
)


def _tc_nodep_body(m0l, m0h, h2l, h2h, hgl, hgh, agl, agh, d0, sB, c3,
                   ol, oh):
    dinv = lax.rsqrt(jnp.maximum(d0[...] + 1.0, 1e-12))
    hg = _cat(hgl[...], hgh[...])
    hb = (_cat(agl[...], agh[...]) * dinv + hg * dinv * dinv) * sB[...] \
        + c3[...]
    npv = _cat(m0l[...], m0h[...]) + _cat(h2l[...], h2h[...]) + hb
    ol[...] = npv[:, :DH]
    oh[...] = npv[:, DH:]


_tc_nodep = pl.pallas_call(
    _tc_nodep_body,
    grid=(_GRID,),
    in_specs=[pl.BlockSpec((_BM, DH), _row)] * 8
    + [pl.BlockSpec((_BM, 1), _row),
       pl.BlockSpec((1, DD), _fix),
       pl.BlockSpec((1, DD), _fix)],
    out_specs=[pl.BlockSpec((_BM, DH), _row), pl.BlockSpec((_BM, DH), _row)],
    out_shape=[jax.ShapeDtypeStruct((NP, DH), jnp.float32),
               jax.ShapeDtypeStruct((NP, DH), jnp.float32)],
)


def _tc_final_body(p, o):
    v = p[...]
    lo = jnp.max(v[:NS], axis=0)
    hi = jnp.max(v[NS:], axis=0)
    o[...] = jnp.concatenate([lo[:NSEG], hi[:NSEG]], axis=1)


_tc_final = pl.pallas_call(
    _tc_final_body,
    grid=(1,),
    in_specs=[pl.BlockSpec((NC * NS, SP, DH), lambda i: (0, 0, 0))],
    out_specs=pl.BlockSpec((NSEG, DD), _fix),
    out_shape=jax.ShapeDtypeStruct((NSEG, DD), jnp.float32),
)

_BNS = float(1.0 / (1.0 + 1e-5) ** 0.5)


def kernel(x, sub_edge_index, node_to_subgraph, params):
    src = sub_edge_index[0]
    dst = sub_edge_index[1]
    xp = jnp.pad(x, ((0, NP - NN), (0, 0)))
    xp_st = jnp.concatenate([xp[:, :DH], xp[:, DH:]], axis=0)
    src_r = src.reshape(NS, ENCH, ECH)
    src2_t = jnp.stack([src_r, src_r + NP])
    dst_t = dst.reshape(NS, ENCH, ECH)
    dst_e = dst.reshape(NC, NS, EPC // ECH, ECH)
    nts_pad = jnp.pad(node_to_subgraph, (0, NP - NN),
                      constant_values=PAD_SEG)
    nts_s = nts_pad.reshape(NS, RNCH, RCH)
    nts_b = jnp.stack([nts_s, nts_s + SP])
    nts_g = nts_pad.reshape(NS, RPT // 16, 16)
    nts_t = nts_pad.reshape(NS, NC, 4, RCH)

    def v(a):
        return a.reshape(1, -1)

    gin_p = []
    for i in range(2):
        p = params["gin%d" % i]
        bn = params["bn%d" % i]
        s1 = p["g1"] * _BNS
        c1 = p["b1"] * s1 + p["be1"]
        s2 = bn["g"] * _BNS
        c2 = p["b2"] * s2 + bn["b"]
        ep = jnp.full((1, DD), 1.0 + p["eps"], jnp.float32)
        gin_p.append((ep, p["W1"], v(s1), v(c1), p["W2"], v(s2), v(c2)))
    vn_p = []
    for i in range(2):
        p = params["vn%d" % i]
        sv1 = p["g1"] * _BNS
        cv1 = p["b1"] * sv1 + p["be1"]
        sv2 = p["g2"] * _BNS
        cv2 = p["b2"] * sv2 + p["be2"]
        vn_p.append((p["W1"], v(sv1), v(cv1), p["W2"], v(sv2), v(cv2)))
    sgl = params["bn_last"]["g"] * _BNS
    c3 = params["gcn"]["b"] * sgl + params["bn_last"]["b"]

    vne_l = jnp.broadcast_to(params["vn_weight"][0, :DH], (SP, DH))
    vne_h = jnp.broadcast_to(params["vn_weight"][0, DH:], (SP, DH))

    sc_edge, sc_seg_sum, sc_bcast, sc_seg_max, sc_deg_cnt = _build_sc()

    deg_st, cnt_parts = sc_deg_cnt(dst_e, nts_t)
    deg_col = (deg_st[:NP, 0] + deg_st[NP:, 0]).reshape(NP, 1)
    c0 = cnt_parts[0, :, 0:1]
    c1 = cnt_parts[1, :, 0:1]

    h_st = xp_st
    m0_l = m0_h = None
    h_l, h_h = xp[:, :DH], xp[:, DH:]
    for i in range(2):
        agg = sc_edge(h_st, src2_t, dst_t)
        ep, w1, s1, cc1, w2, s2, cc2 = gin_p[i]
        m_l, m_h = _tc_gin(h_l, h_h, agg[:NP], agg[NP:],
                           ep, w1, s1, cc1, w2, s2, cc2)
        if i == 0:
            m0_l, m0_h = m_l, m_h
        m_st = jnp.concatenate([m_l, m_h], axis=0)
        sums = sc_seg_sum(m_st, nts_s)
        w1v, sv1, cv1, w2v, sv2, cv2 = vn_p[i]
        vne_l, vne_h = _tc_vn(sums[0], sums[1], c0, c1, vne_l, vne_h,
                              w1v, sv1, cv1, w2v, sv2, cv2)
        vne_st = jnp.concatenate([vne_l, vne_h], axis=0)
        h_st = sc_bcast(m_st, vne_st, nts_b)
        h_l, h_h = h_st[:NP], h_st[NP:]

    hg_l, hg_h, hg2_l, hg2_h = _tc_gcn_pre(h_l, h_h, deg_col,
                                           params["gcn"]["W"])
    hg2_st = jnp.concatenate([hg2_l, hg2_h], axis=0)
    accg = sc_edge(hg2_st, src2_t, dst_t)
    np_l, np_h = _tc_nodep(m0_l, m0_h, h_l, h_h, hg_l, hg_h,
                           accg[:NP], accg[NP:], deg_col, v(sgl), v(c3))
    np_st = jnp.concatenate([np_l, np_h], axis=0)
    parts = sc_seg_max(np_st, nts_g)
    return _tc_final(parts.reshape(NC * NS, SP, DH))
